# baseline (speedup 1.0000x reference)
import dataclasses
import functools

import jax
import jax.numpy as jnp
from jax import lax
from jax.experimental import pallas as pl
from jax.experimental.pallas import tpu as pltpu
from jax.experimental.pallas import tpu_sc as plsc

_L = 16
_NC = 2
_NS = 16
_NW = _NC * _NS
_SLAB = 32
_CB = 1024


@functools.lru_cache(maxsize=None)
def _make_one_hot_sc(batch: int, depth: int):
    nblk = batch // _CB
    nvec = _CB // _L

    mesh = plsc.VectorSubcoreMesh(core_axis_name="c", subcore_axis_name="s")

    cparams = pltpu.CompilerParams()
    if "needs_layout_passes" in pltpu.CompilerParams.__dataclass_fields__:
        cparams = dataclasses.replace(cparams, needs_layout_passes=False)
    if "use_tc_tiling_on_sc" in pltpu.CompilerParams.__dataclass_fields__:
        cparams = dataclasses.replace(cparams, use_tc_tiling_on_sc=True)

    @functools.partial(
        pl.kernel,
        out_type=jax.ShapeDtypeStruct((depth, batch), jnp.float32),
        mesh=mesh,
        compiler_params=cparams,
        scratch_types=[
            pltpu.VMEM((batch,), jnp.int32),
            pltpu.VMEM((_SLAB, _CB), jnp.float32),
            pltpu.VMEM((_SLAB, _CB), jnp.float32),
            pltpu.SemaphoreType.DMA,
            pltpu.SemaphoreType.DMA,
            pltpu.SemaphoreType.DMA,
        ],
    )
    def kern(x_hbm, o_hbm, idx_v, buf0, buf1, sem0, sem1, isem):
        wid = lax.axis_index("s") * _NC + lax.axis_index("c")
        c0 = wid * _SLAB
        c1 = jnp.minimum(c0 + _SLAB, depth)
        slab_sz = (c1 - c0).astype(jnp.uint32)
        idx_cp = pltpu.make_async_copy(x_hbm, idx_v, isem)
        idx_cp.start()

        zeros = jnp.zeros((_L,), jnp.float32)
        ones = jnp.ones((_L,), jnp.float32)
        lane = lax.iota(jnp.int32, _L)
        bufs = (buf0, buf1)
        sems = (sem0, sem1)
        nring = len(bufs)

        def zero_init(buf):
            @pl.loop(0, _SLAB)
            def _(r, buf=buf):
                for c in range(_CB // _L):
                    buf[r, pl.ds(c * _L, _L)] = zeros

        def scan_scatter(buf, blk, val):
            @pl.loop(0, nvec, step=2)
            def _(j, buf=buf, blk=blk, val=val):
                for u in range(2):
                    v = idx_v[pl.ds(blk * _CB + (j + u) * _L, _L)]
                    rows = v - c0
                    in_slab = plsc.bitcast(rows, jnp.uint32) < slab_sz
                    plsc.store_scatter(
                        buf, [rows, (j + u) * _L + lane], val, mask=in_slab)

        def each_dma(buf, sem, blk, fn):
            @pl.when(c0 + _SLAB <= depth)
            def _():
                fn(pltpu.make_async_copy(
                    buf,
                    o_hbm.at[pl.ds(c0, _SLAB), pl.ds(blk * _CB, _CB)],
                    sem,
                ))

            @pl.when(c0 + _SLAB > depth)
            def _():
                for s in range(_SLAB // 8):
                    @pl.when(c0 + 8 * s < depth)
                    def _(s=s):
                        fn(pltpu.make_async_copy(
                            buf.at[pl.ds(8 * s, 8)],
                            o_hbm.at[pl.ds(c0 + 8 * s, 8),
                                     pl.ds(blk * _CB, _CB)],
                            sem,
                        ))

        def fill_and_send(buf, sem, blk):
            scan_scatter(buf, blk, ones)
            each_dma(buf, sem, blk, lambda cp: cp.start())

        def reclaim(buf, sem, blk):
            each_dma(buf, sem, blk, lambda cp: cp.wait())
            scan_scatter(buf, blk, zeros)

        nsteady = ((nblk - nring) // nring) * nring
        for h in range(nring):
            zero_init(bufs[h])
            if h == 0:
                idx_cp.wait()
            fill_and_send(bufs[h], sems[h], h)

        @pl.loop(nring, nring + nsteady, step=nring)
        def _(g):
            for h in range(nring):
                reclaim(bufs[h], sems[h], g + h - nring)
                fill_and_send(bufs[h], sems[h], g + h)

        for blk in range(nring + nsteady, nblk):
            h = blk % nring
            reclaim(bufs[h], sems[h], blk - nring)
            fill_and_send(bufs[h], sems[h], blk)
        for blk in range(nblk - nring, nblk):
            h = blk % nring
            each_dma(bufs[h], sems[h], blk, lambda cp: cp.wait())

    return kern


def kernel(X_in, ones):
    batch = X_in.shape[0]
    depth = ones.shape[0]
    out_t = _make_one_hot_sc(batch, depth)(X_in.astype(jnp.int32))
    return out_t.T


# speedup vs baseline: 1.0124x; 1.0124x over previous
import dataclasses
import functools

import jax
import jax.numpy as jnp
from jax import lax
from jax.experimental import pallas as pl
from jax.experimental.pallas import tpu as pltpu
from jax.experimental.pallas import tpu_sc as plsc

_L = 16
_NC = 2
_NS = 16
_NW = _NC * _NS
_SLAB = 32
_CB = 1024


@functools.lru_cache(maxsize=None)
def _make_one_hot_sc(batch: int, depth: int):
    nblk = batch // _CB
    nvec = _CB // _L

    mesh = plsc.VectorSubcoreMesh(core_axis_name="c", subcore_axis_name="s")

    cparams = pltpu.CompilerParams()
    if "needs_layout_passes" in pltpu.CompilerParams.__dataclass_fields__:
        cparams = dataclasses.replace(cparams, needs_layout_passes=False)
    if "use_tc_tiling_on_sc" in pltpu.CompilerParams.__dataclass_fields__:
        cparams = dataclasses.replace(cparams, use_tc_tiling_on_sc=True)

    @functools.partial(
        pl.kernel,
        out_type=jax.ShapeDtypeStruct((depth, batch), jnp.float32),
        mesh=mesh,
        compiler_params=cparams,
        scratch_types=[
            pltpu.VMEM((batch,), jnp.int32),
            pltpu.VMEM((_SLAB, _CB), jnp.float32),
            pltpu.VMEM((_SLAB, _CB), jnp.float32),
            pltpu.SemaphoreType.DMA,
            pltpu.SemaphoreType.DMA,
            pltpu.SemaphoreType.DMA,
        ],
    )
    def kern(x_hbm, o_hbm, idx_v, buf0, buf1, sem0, sem1, isem):
        wid = lax.axis_index("s") * _NC + lax.axis_index("c")
        c0 = wid * _SLAB
        c1 = jnp.minimum(c0 + _SLAB, depth)
        slab_sz = (c1 - c0).astype(jnp.uint32)
        idx_cp = pltpu.make_async_copy(x_hbm, idx_v, isem)
        idx_cp.start()

        zeros = jnp.zeros((_L,), jnp.float32)
        ones = jnp.ones((_L,), jnp.float32)
        lane = lax.iota(jnp.int32, _L)
        bufs = (buf0, buf1)
        sems = (sem0, sem1)
        nring = len(bufs)

        def zero_init(buf):
            @pl.loop(0, _SLAB)
            def _(r, buf=buf):
                for c in range(_CB // _L):
                    buf[r, pl.ds(c * _L, _L)] = zeros

        def scan_scatter(buf, blk, val):
            @pl.loop(0, nvec)
            def _(j, buf=buf, blk=blk, val=val):
                v = idx_v[pl.ds(blk * _CB + j * _L, _L)]
                rows = v - c0
                in_slab = plsc.bitcast(rows, jnp.uint32) < slab_sz
                plsc.store_scatter(
                    buf, [rows, j * _L + lane], val, mask=in_slab)

        def each_dma(buf, sem, blk, fn):
            for s in range(_SLAB // 8):
                @pl.when(c0 + 8 * s < depth)
                def _(s=s):
                    fn(pltpu.make_async_copy(
                        buf.at[pl.ds(8 * s, 8)],
                        o_hbm.at[pl.ds(c0 + 8 * s, 8),
                                 pl.ds(blk * _CB, _CB)],
                        sem,
                    ))

        def fill_and_send(buf, sem, blk):
            scan_scatter(buf, blk, ones)
            each_dma(buf, sem, blk, lambda cp: cp.start())

        def reclaim(buf, sem, blk):
            each_dma(buf, sem, blk, lambda cp: cp.wait())
            scan_scatter(buf, blk, zeros)

        nsteady = ((nblk - nring) // nring) * nring
        for h in range(nring):
            zero_init(bufs[h])
            if h == 0:
                idx_cp.wait()
            fill_and_send(bufs[h], sems[h], h)

        @pl.loop(nring, nring + nsteady, step=nring)
        def _(g):
            for h in range(nring):
                reclaim(bufs[h], sems[h], g + h - nring)
                fill_and_send(bufs[h], sems[h], g + h)

        for blk in range(nring + nsteady, nblk):
            h = blk % nring
            reclaim(bufs[h], sems[h], blk - nring)
            fill_and_send(bufs[h], sems[h], blk)
        for blk in range(nblk - nring, nblk):
            h = blk % nring
            each_dma(bufs[h], sems[h], blk, lambda cp: cp.wait())

    return kern


def kernel(X_in, ones):
    batch = X_in.shape[0]
    depth = ones.shape[0]
    out_t = _make_one_hot_sc(batch, depth)(X_in.astype(jnp.int32))
    return out_t.T
